# baseline (speedup 1.0000x reference)
import functools

import jax
import jax.numpy as jnp
from jax import lax
from jax.experimental import pallas as pl
from jax.experimental.pallas import tpu as pltpu

B = 4096
D = 64
_NSEM = 8
_CHUNK = B // _NSEM


_HALF = B // 2


def _gather_body(uidx_ref, iidx_ref, utab_ref, itab_ref, uout_ref, iout_ref,
                 uvmem, ivmem, sems, hsems, osem):
    def issue(k, _):
        for j in range(_NSEM):
            row = k * _NSEM + j
            hrow = _HALF + row
            pltpu.make_async_copy(
                utab_ref.at[pl.ds(uidx_ref[row], 1)],
                uvmem.at[pl.ds(row, 1)],
                sems.at[j],
            ).start()
            pltpu.make_async_copy(
                utab_ref.at[pl.ds(uidx_ref[hrow], 1)],
                uout_ref.at[pl.ds(hrow, 1)],
                hsems.at[j],
            ).start()
            pltpu.make_async_copy(
                itab_ref.at[pl.ds(iidx_ref[row], 1)],
                ivmem.at[pl.ds(row, 1)],
                sems.at[j],
            ).start()
            pltpu.make_async_copy(
                itab_ref.at[pl.ds(iidx_ref[hrow], 1)],
                iout_ref.at[pl.ds(hrow, 1)],
                hsems.at[j],
            ).start()
        return 0
    lax.fori_loop(0, _HALF // _NSEM, issue, 0, unroll=True)

    nrows = 2 * (_HALF // _NSEM)
    for j in range(_NSEM):
        pltpu.make_async_copy(
            utab_ref.at[pl.ds(0, nrows)],
            uvmem.at[pl.ds(0, nrows)],
            sems.at[j],
        ).wait()
        pltpu.make_async_copy(
            utab_ref.at[pl.ds(0, nrows)],
            uout_ref.at[pl.ds(0, nrows)],
            hsems.at[j],
        ).wait()

    ucp = pltpu.make_async_copy(uvmem, uout_ref.at[pl.ds(0, _HALF)], osem)
    icp = pltpu.make_async_copy(ivmem, iout_ref.at[pl.ds(0, _HALF)], osem)
    ucp.start()
    icp.start()
    ucp.wait()
    icp.wait()


_gather = pl.pallas_call(
    _gather_body,
    in_specs=[
        pl.BlockSpec(memory_space=pltpu.SMEM),
        pl.BlockSpec(memory_space=pltpu.SMEM),
        pl.BlockSpec(memory_space=pl.ANY),
        pl.BlockSpec(memory_space=pl.ANY),
    ],
    out_specs=[
        pl.BlockSpec(memory_space=pl.ANY),
        pl.BlockSpec(memory_space=pl.ANY),
    ],
    out_shape=[
        jax.ShapeDtypeStruct((B, D), jnp.float32),
        jax.ShapeDtypeStruct((B, D), jnp.float32),
    ],
    scratch_shapes=[
        pltpu.VMEM((_HALF, D), jnp.float32),
        pltpu.VMEM((_HALF, D), jnp.float32),
        pltpu.SemaphoreType.DMA((_NSEM,)),
        pltpu.SemaphoreType.DMA((_NSEM,)),
        pltpu.SemaphoreType.DMA,
    ],
)


_BM = 512
_BN = 1024


def _mm_body(u_ref, i_ref, o_ref):
    o_ref[...] = lax.dot_general(
        u_ref[...], i_ref[...],
        (((1,), (1,)), ((), ())),
        preferred_element_type=jnp.float32,
    )


_matmul = pl.pallas_call(
    _mm_body,
    grid=(B // _BM, B // _BN),
    in_specs=[
        pl.BlockSpec((_BM, D), lambda i, j: (i, 0)),
        pl.BlockSpec((_BN, D), lambda i, j: (j, 0)),
    ],
    out_specs=pl.BlockSpec((_BM, _BN), lambda i, j: (i, j)),
    out_shape=jax.ShapeDtypeStruct((B, B), jnp.float32),
)


@jax.jit
def kernel(user_indices, item_indices, user_table, item_table):
    user_embs, item_embs = _gather(
        user_indices.astype(jnp.int32), item_indices.astype(jnp.int32),
        user_table, item_table)
    return _matmul(user_embs, item_embs)


# speedup vs baseline: 1.0543x; 1.0543x over previous
import functools

import jax
import jax.numpy as jnp
from jax import lax
from jax.experimental import pallas as pl
from jax.experimental.pallas import tpu as pltpu

B = 4096
D = 64
_NSEM = 8
_CHUNK = B // _NSEM


_HALF = B // 2


def _gather_body(uidx_ref, iidx_ref, utab_ref, itab_ref, uout_ref, iout_ref,
                 uvmem, ivmem, sems, hsems, osem):
    def issue(k, _):
        for j in range(_NSEM):
            row = k * _NSEM + j
            hrow = _HALF + row
            pltpu.make_async_copy(
                utab_ref.at[pl.ds(uidx_ref[row], 1)],
                uvmem.at[pl.ds(row, 1)],
                sems.at[j],
            ).start(priority=0)
            pltpu.make_async_copy(
                utab_ref.at[pl.ds(uidx_ref[hrow], 1)],
                uvmem.at[pl.ds(hrow, 1)],
                hsems.at[j],
            ).start(priority=1)
            pltpu.make_async_copy(
                itab_ref.at[pl.ds(iidx_ref[row], 1)],
                ivmem.at[pl.ds(row, 1)],
                sems.at[j],
            ).start(priority=0)
            pltpu.make_async_copy(
                itab_ref.at[pl.ds(iidx_ref[hrow], 1)],
                ivmem.at[pl.ds(hrow, 1)],
                hsems.at[j],
            ).start(priority=1)
        return 0
    lax.fori_loop(0, _HALF // _NSEM, issue, 0, unroll=True)

    nrows = 2 * (_HALF // _NSEM)
    for j in range(_NSEM):
        pltpu.make_async_copy(
            utab_ref.at[pl.ds(0, nrows)],
            uvmem.at[pl.ds(0, nrows)],
            sems.at[j],
        ).wait()
        pltpu.make_async_copy(
            utab_ref.at[pl.ds(0, nrows)],
            uvmem.at[pl.ds(0, nrows)],
            hsems.at[j],
        ).wait()

    ucp = pltpu.make_async_copy(uvmem, uout_ref, osem)
    icp = pltpu.make_async_copy(ivmem, iout_ref, osem)
    ucp.start()
    icp.start()
    ucp.wait()
    icp.wait()


_gather = pl.pallas_call(
    _gather_body,
    in_specs=[
        pl.BlockSpec(memory_space=pltpu.SMEM),
        pl.BlockSpec(memory_space=pltpu.SMEM),
        pl.BlockSpec(memory_space=pl.ANY),
        pl.BlockSpec(memory_space=pl.ANY),
    ],
    out_specs=[
        pl.BlockSpec(memory_space=pl.ANY),
        pl.BlockSpec(memory_space=pl.ANY),
    ],
    out_shape=[
        jax.ShapeDtypeStruct((B, D), jnp.float32),
        jax.ShapeDtypeStruct((B, D), jnp.float32),
    ],
    scratch_shapes=[
        pltpu.VMEM((B, D), jnp.float32),
        pltpu.VMEM((B, D), jnp.float32),
        pltpu.SemaphoreType.DMA((_NSEM,)),
        pltpu.SemaphoreType.DMA((_NSEM,)),
        pltpu.SemaphoreType.DMA,
    ],
)


_BM = 512
_BN = 1024


def _mm_body(u_ref, i_ref, o_ref):
    o_ref[...] = lax.dot_general(
        u_ref[...], i_ref[...],
        (((1,), (1,)), ((), ())),
        preferred_element_type=jnp.float32,
    )


_matmul = pl.pallas_call(
    _mm_body,
    grid=(B // _BM, B // _BN),
    in_specs=[
        pl.BlockSpec((_BM, D), lambda i, j: (i, 0)),
        pl.BlockSpec((_BN, D), lambda i, j: (j, 0)),
    ],
    out_specs=pl.BlockSpec((_BM, _BN), lambda i, j: (i, j)),
    out_shape=jax.ShapeDtypeStruct((B, B), jnp.float32),
)


@jax.jit
def kernel(user_indices, item_indices, user_table, item_table):
    user_embs, item_embs = _gather(
        user_indices.astype(jnp.int32), item_indices.astype(jnp.int32),
        user_table, item_table)
    return _matmul(user_embs, item_embs)


# speedup vs baseline: 1.0588x; 1.0043x over previous
import functools

import jax
import jax.numpy as jnp
from jax import lax
from jax.experimental import pallas as pl
from jax.experimental.pallas import tpu as pltpu

B = 4096
D = 64
_BM = 512
_BN = 1024
_NI = B // _BM
_NJ = B // _BN


def _body(uidx_ref, iidx_ref, utab_ref, itab_ref, o_ref,
          uvmem, ivmem, usems, isems):
    i = pl.program_id(0)
    j = pl.program_id(1)

    @pl.when(jnp.logical_and(i == 0, j == 0))
    def _issue():
        def u_rows(blk):
            def go(k, _):
                row = blk * _BM + k
                pltpu.make_async_copy(
                    utab_ref.at[pl.ds(uidx_ref[row], 1)],
                    uvmem.at[pl.ds(row, 1)],
                    usems.at[blk],
                ).start()
                return 0
            lax.fori_loop(0, _BM, go, 0, unroll=True)

        def i_rows(blk):
            def go(k, _):
                row = blk * _BN + k
                pltpu.make_async_copy(
                    itab_ref.at[pl.ds(iidx_ref[row], 1)],
                    ivmem.at[pl.ds(row, 1)],
                    isems.at[blk],
                ).start()
                return 0
            lax.fori_loop(0, _BN, go, 0, unroll=True)

        u_rows(0)
        for jb in range(_NJ):
            i_rows(jb)
        for ib in range(1, _NI):
            u_rows(ib)

    @pl.when(j == 0)
    def _wait_u():
        pltpu.make_async_copy(
            utab_ref.at[pl.ds(0, _BM)], uvmem.at[pl.ds(0, _BM)], usems.at[i]
        ).wait()

    @pl.when(i == 0)
    def _wait_i():
        pltpu.make_async_copy(
            itab_ref.at[pl.ds(0, _BN)], ivmem.at[pl.ds(0, _BN)], isems.at[j]
        ).wait()

    u = uvmem[pl.ds(i * _BM, _BM), :]
    v = ivmem[pl.ds(j * _BN, _BN), :]
    o_ref[...] = lax.dot_general(
        u, v, (((1,), (1,)), ((), ())), preferred_element_type=jnp.float32)


_fused = pl.pallas_call(
    _body,
    grid=(_NI, _NJ),
    in_specs=[
        pl.BlockSpec(memory_space=pltpu.SMEM),
        pl.BlockSpec(memory_space=pltpu.SMEM),
        pl.BlockSpec(memory_space=pl.ANY),
        pl.BlockSpec(memory_space=pl.ANY),
    ],
    out_specs=pl.BlockSpec((_BM, _BN), lambda i, j: (i, j)),
    out_shape=jax.ShapeDtypeStruct((B, B), jnp.float32),
    scratch_shapes=[
        pltpu.VMEM((B, D), jnp.float32),
        pltpu.VMEM((B, D), jnp.float32),
        pltpu.SemaphoreType.DMA((_NI,)),
        pltpu.SemaphoreType.DMA((_NJ,)),
    ],
)


@jax.jit
def kernel(user_indices, item_indices, user_table, item_table):
    return _fused(
        user_indices.astype(jnp.int32), item_indices.astype(jnp.int32),
        user_table, item_table)
